# baseline (speedup 1.0000x reference)
import jax
import jax.numpy as jnp
from jax.experimental import pallas as pl
from jax.experimental.pallas import tpu as pltpu

DIM = 512
CHUNK = 4096
NBUF = 4
BPC = 4


def _pool_kernel(b_ref, x_hbm, w_ref, out_ref, xbuf, acc_ref, ssum_ref, sems):
    p = pl.program_id(0)
    n, d = x_hbm.shape[1], x_hbm.shape[2]
    ncs = n // CHUNK
    total = BPC * ncs

    def copy_for(c):
        gb = p * BPC + c // ncs
        off = (c % ncs) * CHUNK
        slot = c % NBUF
        return pltpu.make_async_copy(
            x_hbm.at[gb, pl.ds(off, CHUNK), :],
            xbuf.at[slot],
            sems.at[slot],
        )

    def start_copy(c):
        copy_for(c).start()

    def wait_copy(c):
        copy_for(c).wait()

    for c in range(min(NBUF - 1, total)):
        start_copy(c)

    acc_ref[...] = jnp.zeros_like(acc_ref)
    for i in range(BPC):
        ssum_ref[i, 0] = 0.0

    def body(c, carry):
        b = c // ncs

        @pl.when(c + NBUF - 1 < total)
        def _():
            start_copy(c + NBUF - 1)

        wait_copy(c)
        x = xbuf[c % NBUF]
        proj = jnp.dot(x, w_ref[...], preferred_element_type=jnp.float32)
        a = jnp.tanh(proj[:, 0:1] + b_ref[0, 0])
        g = jax.nn.sigmoid(proj[:, 1:2] + b_ref[0, 1])
        e = jnp.exp(a * g)
        acc_ref[pl.ds(b, 1), :] += jnp.sum(e * x, axis=0, keepdims=True)
        ssum_ref[b, 0] += jnp.sum(e)
        return carry

    jax.lax.fori_loop(0, total, body, 0)

    for i in range(BPC):
        out_ref[i, 0, :] = acc_ref[i, :] * (1.0 / ssum_ref[i, 0])


def kernel(x, W_a, b_a, W_g, b_g):
    B, N, D = x.shape
    w = jnp.concatenate([W_a, W_g], axis=1)
    biases = jnp.stack([b_a[0], b_g[0]]).reshape(1, 2)

    out = pl.pallas_call(
        _pool_kernel,
        grid=(B // BPC,),
        in_specs=[
            pl.BlockSpec(memory_space=pltpu.SMEM),
            pl.BlockSpec(memory_space=pl.ANY),
            pl.BlockSpec((D, 2), lambda p: (0, 0)),
        ],
        out_specs=pl.BlockSpec((BPC, 1, D), lambda p: (p, 0, 0)),
        out_shape=jax.ShapeDtypeStruct((B, 1, D), jnp.float32),
        scratch_shapes=[
            pltpu.VMEM((NBUF, CHUNK, DIM), jnp.float32),
            pltpu.VMEM((BPC, DIM), jnp.float32),
            pltpu.SMEM((BPC, 1), jnp.float32),
            pltpu.SemaphoreType.DMA((NBUF,)),
        ],
        compiler_params=pltpu.CompilerParams(
            dimension_semantics=("parallel",),
        ),
    )(biases, x, w)
    return out
